# Initial kernel scaffold; baseline (speedup 1.0000x reference)
import functools

import jax
import jax.numpy as jnp
from jax import lax
from jax.experimental import pallas as pl
from jax.experimental.pallas import tpu as pltpu
from jax.experimental.pallas import tpu_sc as plsc

N_NODES = 100000
N_EDGES = 3200000
D_NODE = 128
D_EDGE = 16
D_OUT = 128

NC = 2
NS = 16
NW = NC * NS

E_W = N_EDGES // NW
B = 80
K = 10
CHUNK = B * K
NJ = E_W // CHUNK
ROWS_S = N_NODES // NS
ZROWS = 625


def _sc_segment_sum(edges, recv):
    mesh = plsc.VectorSubcoreMesh(core_axis_name="c", subcore_axis_name="s")

    @functools.partial(
        pl.kernel,
        mesh=mesh,
        out_type=jax.ShapeDtypeStruct((NC, N_NODES, D_EDGE), jnp.float32),
        scratch_types=[
            pltpu.VMEM((K, B), jnp.int32),
            pltpu.VMEM((K, B, D_EDGE), jnp.float32),
            pltpu.VMEM((ZROWS, D_EDGE), jnp.float32),
            pltpu.VMEM_SHARED((N_NODES, D_EDGE), jnp.float32),
        ],
    )
    def k(edges_hbm, recv_hbm, out_hbm, ibuf, ebuf, zbuf, acc):
        c = lax.axis_index("c")
        s = lax.axis_index("s")
        wid = s * NC + c

        def zero_body(i, carry):
            zbuf[i] = jnp.zeros((D_EDGE,), jnp.float32)
            return carry

        lax.fori_loop(0, ZROWS, zero_body, 0)
        base = s * ROWS_S
        for t in range(ROWS_S // ZROWS):
            pltpu.sync_copy(zbuf, acc.at[pl.ds(base + t * ZROWS, ZROWS)])
        plsc.subcore_barrier()

        def jloop(j, carry):
            pltpu.sync_copy(recv_hbm.at[wid, j], ibuf)
            pltpu.sync_copy(edges_hbm.at[wid, j], ebuf)
            for kk in range(K):
                pltpu.sync_copy(ebuf.at[kk], acc.at[ibuf.at[kk]], add=True)
            return carry

        lax.fori_loop(0, NJ, jloop, 0)
        plsc.subcore_barrier()

        pltpu.sync_copy(acc.at[pl.ds(base, ROWS_S)],
                        out_hbm.at[c, pl.ds(base, ROWS_S)])

    return k(edges, recv)


_R = 1000


def _mm_body(parts_ref, nodes_ref, w_ref, b_ref, out_ref):
    agg = parts_ref[0] + parts_ref[1]
    out_ref[...] = (
        jnp.dot(agg, w_ref[0:D_EDGE, :], preferred_element_type=jnp.float32)
        + jnp.dot(nodes_ref[...], w_ref[D_EDGE:, :],
                  preferred_element_type=jnp.float32)
        + b_ref[...]
    )


def _tc_matmul(parts, nodes, W, b):
    grid = (N_NODES // _R,)
    return pl.pallas_call(
        _mm_body,
        grid=grid,
        in_specs=[
            pl.BlockSpec((NC, _R, D_EDGE), lambda i: (0, i, 0)),
            pl.BlockSpec((_R, D_NODE), lambda i: (i, 0)),
            pl.BlockSpec((D_EDGE + D_NODE, D_OUT), lambda i: (0, 0)),
            pl.BlockSpec((1, D_OUT), lambda i: (0, 0)),
        ],
        out_specs=pl.BlockSpec((_R, D_OUT), lambda i: (i, 0)),
        out_shape=jax.ShapeDtypeStruct((N_NODES, D_OUT), jnp.float32),
    )(parts, nodes, W, b.reshape(1, D_OUT))


def kernel(nodes, edge_attr, senders, receivers, W, b):
    del senders
    recv = receivers.astype(jnp.int32).reshape(NW, NJ, K, B)
    edges = edge_attr.reshape(NW, NJ, K, B, D_EDGE)
    parts = _sc_segment_sum(edges, recv)
    return _tc_matmul(parts, nodes, W, b)


# speedup vs baseline: 7.4952x; 7.4952x over previous
import functools

import jax
import jax.numpy as jnp
from jax import lax
from jax.experimental import pallas as pl
from jax.experimental.pallas import tpu as pltpu
from jax.experimental.pallas import tpu_sc as plsc

N_NODES = 100000
N_EDGES = 3200000
D_NODE = 128
D_EDGE = 16
D_OUT = 128

NC = 2
NS = 16
NW = NC * NS

CHUNK = 512
NCH = N_EDGES // CHUNK
CH_BASE = NCH // NW
CH_REM = NCH % NW
B = 128
PITCH = CHUNK + 1

N_PAD = 102400
ROWS_S = N_PAD // NS
ZROWS = 320


def _sc_segment_sum(ea_t, recv):
    mesh = plsc.VectorSubcoreMesh(core_axis_name="c", subcore_axis_name="s")

    @functools.partial(
        pl.kernel,
        mesh=mesh,
        compiler_params=pltpu.CompilerParams(needs_layout_passes=False,
                                             use_tc_tiling_on_sc=False),
        out_type=jax.ShapeDtypeStruct((NC, N_PAD, D_EDGE), jnp.float32),
        scratch_types=[
            pltpu.VMEM((CHUNK // B, B), jnp.int32),
            pltpu.VMEM((D_EDGE, PITCH), jnp.float32),
            pltpu.VMEM((CHUNK, D_EDGE), jnp.float32),
            pltpu.VMEM((ZROWS, D_EDGE), jnp.float32),
            pltpu.VMEM_SHARED((N_PAD, D_EDGE), jnp.float32),
        ],
    )
    def k(ea_hbm, recv_hbm, out_hbm, ibuf, fbuf, ebuf, zbuf, acc):
        c = lax.axis_index("c")
        s = lax.axis_index("s")
        wid = s * NC + c
        lanes = lax.broadcasted_iota(jnp.int32, (D_EDGE,), 0)

        def zero_body(i, carry):
            zbuf[i] = jnp.zeros((D_EDGE,), jnp.float32)
            return carry

        lax.fori_loop(0, ZROWS, zero_body, 0)
        base = s * ROWS_S
        for t in range(ROWS_S // ZROWS):
            pltpu.sync_copy(zbuf, acc.at[pl.ds(base + t * ZROWS, ZROWS)])
        plsc.subcore_barrier()

        lo = wid * CH_BASE + jnp.minimum(wid, CH_REM)
        hi = lo + CH_BASE + jnp.where(wid < CH_REM, 1, 0)

        def chunk_body(ch, carry):
            pltpu.sync_copy(ea_hbm.at[:, pl.ds(ch * CHUNK, CHUNK)],
                            fbuf.at[:, pl.ds(0, CHUNK)])
            pltpu.sync_copy(recv_hbm.at[ch], ibuf)

            def tr_body(j, carry2):
                cols = jnp.full((D_EDGE,), j * D_EDGE, jnp.int32)
                for i in range(D_EDGE):
                    v = plsc.load_gather(fbuf, [lanes, cols + i])
                    ebuf[j * D_EDGE + i] = v
                return carry2

            lax.fori_loop(0, CHUNK // D_EDGE, tr_body, 0)

            for r in range(CHUNK // B):
                pltpu.sync_copy(
                    ebuf.at[pl.ds(r * B, B)],
                    acc.at[ibuf.at[r]],
                    add=True,
                )
            return carry

        lax.fori_loop(lo, hi, chunk_body, 0)
        plsc.subcore_barrier()

        for t in range(ROWS_S // ZROWS):
            off = base + t * ZROWS
            pltpu.sync_copy(acc.at[pl.ds(off, ZROWS)], zbuf)
            pltpu.sync_copy(zbuf, out_hbm.at[c, pl.ds(off, ZROWS)])

    return k(ea_t, recv)


_R = 1000


def _mm_body(parts_ref, nodes_ref, w_ref, b_ref, out_ref):
    agg = parts_ref[0] + parts_ref[1]
    out_ref[...] = (
        jnp.dot(agg, w_ref[0:D_EDGE, :], preferred_element_type=jnp.float32)
        + jnp.dot(nodes_ref[...], w_ref[D_EDGE:, :],
                  preferred_element_type=jnp.float32)
        + b_ref[...]
    )


def _tc_matmul(parts, nodes, W, b):
    grid = (N_NODES // _R,)
    return pl.pallas_call(
        _mm_body,
        grid=grid,
        in_specs=[
            pl.BlockSpec((NC, _R, D_EDGE), lambda i: (0, i, 0)),
            pl.BlockSpec((_R, D_NODE), lambda i: (i, 0)),
            pl.BlockSpec((D_EDGE + D_NODE, D_OUT), lambda i: (0, 0)),
            pl.BlockSpec((1, D_OUT), lambda i: (0, 0)),
        ],
        out_specs=pl.BlockSpec((_R, D_OUT), lambda i: (i, 0)),
        out_shape=jax.ShapeDtypeStruct((N_NODES, D_OUT), jnp.float32),
    )(parts, nodes, W, b.reshape(1, D_OUT))


def kernel(nodes, edge_attr, senders, receivers, W, b):
    del senders
    ea_t = edge_attr.T
    recv = receivers.astype(jnp.int32).reshape(NCH, CHUNK // B, B)
    parts = _sc_segment_sum(ea_t, recv)
    return _tc_matmul(parts, nodes, W, b)


# speedup vs baseline: 10.8023x; 1.4412x over previous
import functools

import jax
import jax.numpy as jnp
from jax import lax
from jax.experimental import pallas as pl
from jax.experimental.pallas import tpu as pltpu
from jax.experimental.pallas import tpu_sc as plsc

N_NODES = 100000
N_EDGES = 3200000
D_NODE = 128
D_EDGE = 16
D_OUT = 128

NC = 2
NS = 16
NW = NC * NS

CHUNK = 256
NCH = N_EDGES // CHUNK
CH_MAIN = 388
NITER = CH_MAIN // 4
CH_W = 390
CH_LEFT = NCH - CH_W * NW
B = 128
PITCH = CHUNK + 1

N_PAD = 102400
ROWS_S = N_PAD // NS
WSTEPS = ROWS_S // CHUNK


def _sc_segment_sum(ea_t, recv):
    mesh = plsc.VectorSubcoreMesh(core_axis_name="c", subcore_axis_name="s")

    @functools.partial(
        pl.kernel,
        mesh=mesh,
        compiler_params=pltpu.CompilerParams(needs_layout_passes=False,
                                             use_tc_tiling_on_sc=False),
        out_type=jax.ShapeDtypeStruct((NC, N_PAD, D_EDGE), jnp.float32),
        scratch_types=[
            pltpu.VMEM((D_EDGE, PITCH), jnp.float32),
            pltpu.VMEM((D_EDGE, PITCH), jnp.float32),
            pltpu.VMEM((CHUNK, D_EDGE), jnp.float32),
            pltpu.VMEM((CHUNK, D_EDGE), jnp.float32),
            pltpu.VMEM((4, CHUNK // B, B), jnp.int32),
            pltpu.VMEM_SHARED((N_PAD, D_EDGE), jnp.float32),
            pltpu.SemaphoreType.DMA,
            pltpu.SemaphoreType.DMA,
            pltpu.SemaphoreType.DMA,
            pltpu.SemaphoreType.DMA,
        ],
    )
    def k(ea_hbm, recv_hbm, out_hbm, fbuf0, fbuf1, ebuf0, ebuf1, ibuf, acc,
          fsem0, fsem1, ssem0, ssem1):
        c = lax.axis_index("c")
        s = lax.axis_index("s")
        wid = s * NC + c
        lanes = lax.broadcasted_iota(jnp.int32, (D_EDGE,), 0)
        fbuf = (fbuf0, fbuf1)
        ebuf = (ebuf0, ebuf1)
        fsem = (fsem0, fsem1)
        ssem = (ssem0, ssem1)

        lo = wid * CH_W

        def fire_fetch(ch, b, q):
            pltpu.async_copy(ea_hbm.at[:, pl.ds(ch * CHUNK, CHUNK)],
                             fbuf[b].at[:, pl.ds(0, CHUNK)], fsem[b])
            pltpu.async_copy(recv_hbm.at[ch], ibuf.at[q], fsem[b])

        def wait_fetch(b):
            pltpu.make_async_copy(ea_hbm.at[:, pl.ds(0, CHUNK)],
                                  fbuf[b].at[:, pl.ds(0, CHUNK)],
                                  fsem[b]).wait()
            pltpu.make_async_copy(recv_hbm.at[0], ibuf.at[0], fsem[b]).wait()

        def fire_scatter(b, q):
            for r in range(CHUNK // B):
                pltpu.async_copy(ebuf[b].at[pl.ds(r * B, B)],
                                 acc.at[ibuf.at[q, r]], ssem[b], add=True)

        def wait_scatter(b):
            for r in range(CHUNK // B):
                pltpu.make_async_copy(ebuf[b].at[pl.ds(0, B)],
                                      acc.at[ibuf.at[0, 0]], ssem[b]).wait()

        def transpose(b):
            def tr_body(j, carry):
                cols = jnp.full((D_EDGE,), j * D_EDGE, jnp.int32)
                for i in range(D_EDGE):
                    v = plsc.load_gather(fbuf[b], [lanes, cols + i])
                    ebuf[b][j * D_EDGE + i] = v
                return carry

            lax.fori_loop(0, CHUNK // D_EDGE, tr_body, 0)

        fire_fetch(lo, 0, 0)
        fire_fetch(lo + 1, 1, 1)

        def zero_body(i, carry):
            ebuf0[i] = jnp.zeros((D_EDGE,), jnp.float32)
            return carry

        lax.fori_loop(0, CHUNK, zero_body, 0)
        base = s * ROWS_S
        for t in range(WSTEPS):
            pltpu.sync_copy(ebuf0, acc.at[pl.ds(base + t * CHUNK, CHUNK)])
        plsc.subcore_barrier()

        def chunk_body(it, carry):
            ch0 = lo + it * 4
            for q in range(4):
                b = q % 2
                wait_fetch(b)
                if q >= 2:
                    wait_scatter(b)
                else:
                    @pl.when(it > 0)
                    def _():
                        wait_scatter(b)
                transpose(b)
                fire_scatter(b, q)
                fire_fetch(ch0 + q + 2, b, (q + 2) % 4)
            return carry

        lax.fori_loop(0, NITER, chunk_body, 0)

        for q in range(2):
            b = q
            wait_fetch(b)
            wait_scatter(b)
            transpose(b)
            fire_scatter(b, q)
        wait_scatter(0)
        wait_scatter(1)

        @pl.when(wid < CH_LEFT)
        def _():
            ch = NW * CH_W + wid
            pltpu.sync_copy(ea_hbm.at[:, pl.ds(ch * CHUNK, CHUNK)],
                            fbuf0.at[:, pl.ds(0, CHUNK)])
            pltpu.sync_copy(recv_hbm.at[ch], ibuf.at[0])
            transpose(0)
            for r in range(CHUNK // B):
                pltpu.sync_copy(ebuf0.at[pl.ds(r * B, B)],
                                acc.at[ibuf.at[0, r]], add=True)

        plsc.subcore_barrier()

        for t in range(WSTEPS):
            b = t % 2
            if t >= 2:
                pltpu.make_async_copy(ebuf[b],
                                      out_hbm.at[c, pl.ds(0, CHUNK)],
                                      fsem[b]).wait()
            off = base + t * CHUNK
            pltpu.sync_copy(acc.at[pl.ds(off, CHUNK)], ebuf[b])
            pltpu.async_copy(ebuf[b], out_hbm.at[c, pl.ds(off, CHUNK)],
                             fsem[b])
        for b in range(2):
            pltpu.make_async_copy(ebuf[b], out_hbm.at[c, pl.ds(0, CHUNK)],
                                  fsem[b]).wait()

    return k(ea_t, recv)


_R = 1000


def _mm_body(parts_ref, nodes_ref, w_ref, b_ref, out_ref):
    agg = parts_ref[0] + parts_ref[1]
    out_ref[...] = (
        jnp.dot(agg, w_ref[0:D_EDGE, :], preferred_element_type=jnp.float32)
        + jnp.dot(nodes_ref[...], w_ref[D_EDGE:, :],
                  preferred_element_type=jnp.float32)
        + b_ref[...]
    )


def _tc_matmul(parts, nodes, W, b):
    grid = (N_NODES // _R,)
    return pl.pallas_call(
        _mm_body,
        grid=grid,
        in_specs=[
            pl.BlockSpec((NC, _R, D_EDGE), lambda i: (0, i, 0)),
            pl.BlockSpec((_R, D_NODE), lambda i: (i, 0)),
            pl.BlockSpec((D_EDGE + D_NODE, D_OUT), lambda i: (0, 0)),
            pl.BlockSpec((1, D_OUT), lambda i: (0, 0)),
        ],
        out_specs=pl.BlockSpec((_R, D_OUT), lambda i: (i, 0)),
        out_shape=jax.ShapeDtypeStruct((N_NODES, D_OUT), jnp.float32),
    )(parts, nodes, W, b.reshape(1, D_OUT))


def kernel(nodes, edge_attr, senders, receivers, W, b):
    del senders
    ea_t = edge_attr.T
    recv = receivers.astype(jnp.int32).reshape(NCH, CHUNK // B, B)
    parts = _sc_segment_sum(ea_t, recv)
    return _tc_matmul(parts, nodes, W, b)
